# Initial kernel scaffold; baseline (speedup 1.0000x reference)
import jax
import jax.numpy as jnp
from jax.experimental import pallas as pl


def kernel(flat, segment_ids, W_gate, b_gate, W_out, b_out):
    raise NotImplementedError("write your pallas kernel here")


# speedup vs baseline: 4.5694x; 4.5694x over previous
import functools

import jax
import jax.numpy as jnp
from jax import lax
from jax.experimental import pallas as pl
from jax.experimental.pallas import tpu as pltpu

_B = 16


def _tc_body(flat_ref, segs_ref, wg_ref, bg_ref, wo_ref, bo_ref, out_ref,
             A, m, s, *, T, G):
    i = pl.program_id(0)

    @pl.when(i == 0)
    def _init():
        A[...] = jnp.zeros_like(A)
        m[...] = jnp.full_like(m, -jnp.inf)
        s[...] = jnp.zeros_like(s)

    flat_c = flat_ref[...]
    segs2 = segs_ref[0]
    onehot = (lax.broadcasted_iota(jnp.int32, (_B, T), 0) == segs2)

    logits_t = lax.dot_general(
        wg_ref[...], flat_c, (((0,), (1,)), ((), ())),
        preferred_element_type=jnp.float32) + bg_ref[0, 0]

    neg_inf = jnp.float32(-jnp.inf)
    masked = jnp.where(onehot, logits_t, neg_inf)
    m_c = jnp.max(masked, axis=1, keepdims=True)
    m_old = m[...]
    m_new = jnp.maximum(m_old, m_c)
    alpha = jnp.where(m_old == neg_inf, 0.0, jnp.exp(m_old - m_new))

    m_sel = jnp.max(jnp.where(onehot, m_new, neg_inf), axis=0, keepdims=True)
    e_t = jnp.exp(logits_t - m_sel)
    we = jnp.where(onehot, e_t, 0.0)
    s_c = jnp.sum(we, axis=1, keepdims=True)
    A_c = jnp.dot(we, flat_c, preferred_element_type=jnp.float32)

    A[...] = A[...] * alpha + A_c
    s[...] = s[...] * alpha + s_c
    m[...] = m_new

    @pl.when(i == G - 1)
    def _fin():
        S = s[...]
        denom = S + 1e-16
        pooled = jnp.dot(A[...] / denom, wo_ref[...],
                         preferred_element_type=jnp.float32)
        out_ref[...] = pooled + bo_ref[...] * (S / denom)


def kernel(flat, segment_ids, W_gate, b_gate, W_out, b_out):
    N, D = flat.shape
    T = 1024
    G = N // T
    segs = segment_ids.astype(jnp.int32).reshape(G, 1, T)
    out = pl.pallas_call(
        functools.partial(_tc_body, T=T, G=G),
        grid=(G,),
        in_specs=[
            pl.BlockSpec((T, D), lambda i: (i, 0)),
            pl.BlockSpec((1, 1, T), lambda i: (i, 0, 0)),
            pl.BlockSpec((D, 1), lambda i: (0, 0)),
            pl.BlockSpec((1, 1), lambda i: (0, 0)),
            pl.BlockSpec((D, 2), lambda i: (0, 0)),
            pl.BlockSpec((1, 2), lambda i: (0, 0)),
        ],
        out_specs=pl.BlockSpec((_B, 2), lambda i: (0, 0)),
        out_shape=jax.ShapeDtypeStruct((_B, 2), jnp.float32),
        scratch_shapes=[
            pltpu.VMEM((_B, D), jnp.float32),
            pltpu.VMEM((_B, 1), jnp.float32),
            pltpu.VMEM((_B, 1), jnp.float32),
        ],
        compiler_params=pltpu.CompilerParams(
            dimension_semantics=("arbitrary",)),
    )(flat, segs, W_gate, b_gate.reshape(1, 1), W_out, b_out.reshape(1, 2))
    return out
